# baseline (speedup 1.0000x reference)
import functools

import jax
import jax.numpy as jnp
from jax import lax
from jax.experimental import pallas as pl
from jax.experimental.pallas import tpu as pltpu
from jax.experimental.pallas import tpu_sc as plsc

N = 10000
E = 320000
F = 128
HID = 256
DO = 64
NC, NS = 2, 16
NW = NC * NS
EPW = E // NW
CH = 80
NCHUNK = EPW // CH
RB = 1000
NB = N // RB

_f32 = jnp.float32


def _mesh():
    return plsc.VectorSubcoreMesh(
        core_axis_name="c", subcore_axis_name="s", num_cores=NC, num_subcores=NS
    )


@functools.lru_cache(maxsize=None)
def _sc_degree_kernel():
    @functools.partial(
        pl.kernel,
        out_type=[jax.ShapeDtypeStruct((NC, 1, 2 * N), _f32)] * 6,
        mesh=_mesh(),
        scratch_types=(
            [pltpu.VMEM_SHARED((2 * N,), _f32)] * 6
            + [
                pltpu.VMEM((CH,), _f32),
                pltpu.VMEM((NCHUNK, CH), jnp.int32),
                pltpu.VMEM((NCHUNK, CH), jnp.int32),
                pltpu.SemaphoreType.DMA,
            ]
        ),
    )
    def k(*refs):
        idx_hbms = refs[0:12]
        ones_hbm, z_hbm = refs[12], refs[13]
        outs = refs[14:20]
        accs = refs[20:26]
        ones, ia_b, ib_b, sem = refs[26:30]
        c = lax.axis_index("c")
        s = lax.axis_index("s")
        wid = s * NC + c

        for v in range(6):
            @pl.when(s == v)
            def _(v=v):
                pltpu.sync_copy(z_hbm, accs[v])

        pltpu.sync_copy(ones_hbm, ones)
        plsc.subcore_barrier()

        for v in range(6):
            acc = accs[v]
            pltpu.sync_copy(idx_hbms[2 * v].at[wid], ia_b)
            pltpu.sync_copy(idx_hbms[2 * v + 1].at[wid], ib_b)

            def _wait_pair():
                pltpu.make_async_copy(ones, acc.at[ia_b.at[0]], sem).wait()
                pltpu.make_async_copy(ones, acc.at[ib_b.at[0]], sem).wait()

            @pl.loop(0, NCHUNK)
            def _(j, acc=acc, _wait_pair=_wait_pair):
                pltpu.async_copy(ones, acc.at[ia_b.at[j]], sem, add=True)
                pltpu.async_copy(ones, acc.at[ib_b.at[j]], sem, add=True)

                @pl.when(j > 0)
                def _():
                    _wait_pair()

            _wait_pair()

        plsc.subcore_barrier()

        for v in range(6):
            @pl.when(s == v)
            def _(v=v):
                pltpu.sync_copy(accs[v], outs[v].at[c, 0])

    return k


@functools.lru_cache(maxsize=None)
def _sc_segsum_kernel(nv, sidecar):
    out_type = [jax.ShapeDtypeStruct((NC, N, F), _f32)] * nv
    scratch = [
        pltpu.VMEM_SHARED((N, F), _f32),
        pltpu.VMEM((CH, F), _f32),
        pltpu.VMEM((CH, F), _f32),
        pltpu.VMEM((EPW,), jnp.int32),
        pltpu.VMEM((1, CH), jnp.int32),
        pltpu.VMEM((1, CH), jnp.int32),
        pltpu.SemaphoreType.DMA,
        pltpu.SemaphoreType.DMA,
        pltpu.SemaphoreType.DMA,
        pltpu.SemaphoreType.DMA,
    ]
    if sidecar:
        out_type += [jax.ShapeDtypeStruct((NC, 1, N), _f32)] * nv
        scratch += [
            pltpu.VMEM_SHARED((N,), _f32),
            pltpu.VMEM((CH,), _f32),
            pltpu.VMEM((CH,), _f32),
            pltpu.VMEM((N,), _f32),
        ]
    npv = 4 if sidecar else 3

    def body(*refs):
        n_in = npv * nv + 1 + (1 if sidecar else 0)
        ins = refs[0:n_in]
        outs = refs[n_in:n_in + (2 if sidecar else 1) * nv]
        scr = refs[n_in + len(outs):]
        if sidecar:
            (acc, rows_a, rows_b, gidx, sidx_a, sidx_b,
             sem_a, sem_b, sem_ia, sem_ib,
             accs, rows_sa, rows_sb, xs_v) = scr
        else:
            (acc, rows_a, rows_b, gidx, sidx_a, sidx_b,
             sem_a, sem_b, sem_ia, sem_ib) = scr
            rows_sa = rows_sb = xs_v = accs = None
        c = lax.axis_index("c")
        s = lax.axis_index("s")
        wid = s * NC + c
        srow0 = wid * NCHUNK

        z_hbm = ins[npv * nv]
        z1_hbm = ins[npv * nv + 1] if sidecar else None

        def gstart(xm_hbm, j, rows, sem):
            pltpu.async_copy(
                xm_hbm.at[gidx.at[pl.ds(j * CH, CH)]], rows, sem
            )

        def gwait(xm_hbm, j, rows, sem):
            pltpu.make_async_copy(
                xm_hbm.at[gidx.at[pl.ds(j * CH, CH)]], rows, sem
            ).wait()

        for v in range(nv):
            if sidecar:
                xm_hbm = ins[4 * v + 0]
                xs_hbm = ins[4 * v + 1]
                gids_hbm = ins[4 * v + 2]
                sids_hbm = ins[4 * v + 3]
                out_hbm = outs[v]
                outs_hbm = outs[nv + v]
            else:
                xm_hbm = ins[3 * v + 0]
                gids_hbm = ins[3 * v + 1]
                sids_hbm = ins[3 * v + 2]
                out_hbm = outs[v]
                outs_hbm = None

            @pl.when(s < 10)
            def _(v=v):
                pltpu.sync_copy(
                    z_hbm.at[pl.ds(s * RB, RB)], acc.at[pl.ds(s * RB, RB)]
                )

            if sidecar:
                @pl.when(s == 10)
                def _(v=v):
                    pltpu.sync_copy(z1_hbm, accs)

                pltpu.sync_copy(xs_hbm, xs_v)

            pltpu.sync_copy(gids_hbm.at[wid], gidx)
            plsc.subcore_barrier()

            def sistart(j, sbuf, sem, sids_hbm=sids_hbm):
                pltpu.async_copy(
                    sids_hbm.at[pl.ds(srow0 + j, 1)], sbuf, sem
                )

            def siwait(sbuf, sem, sids_hbm=sids_hbm):
                pltpu.make_async_copy(
                    sids_hbm.at[pl.ds(srow0, 1)], sbuf, sem
                ).wait()

            def scatter(j, rows, sbuf, rows_s):
                if sidecar:
                    for r in range(CH // 16):
                        gi = gidx[pl.ds(j * CH + r * 16, 16)]
                        rows_s[pl.ds(r * 16, 16)] = plsc.load_gather(
                            xs_v, [gi]
                        )
                pltpu.sync_copy(rows, acc.at[sbuf.at[0]], add=True)
                if sidecar:
                    pltpu.sync_copy(rows_s, accs.at[sbuf.at[0]], add=True)

            gs = functools.partial(gstart, xm_hbm)
            gw = functools.partial(gwait, xm_hbm)

            sistart(0, sidx_a, sem_ia)
            sistart(1, sidx_b, sem_ib)
            gs(0, rows_a, sem_a)

            @pl.loop(0, (NCHUNK - 1) // 2)
            def _(jj, gs=gs, gw=gw, sistart=sistart, siwait=siwait,
                  scatter=scatter):
                j0 = 2 * jj
                gs(j0 + 1, rows_b, sem_b)
                gw(j0, rows_a, sem_a)
                siwait(sidx_a, sem_ia)
                scatter(j0, rows_a, sidx_a, rows_sa)
                sistart(j0 + 2, sidx_a, sem_ia)
                gs(j0 + 2, rows_a, sem_a)
                gw(j0 + 1, rows_b, sem_b)
                siwait(sidx_b, sem_ib)
                scatter(j0 + 1, rows_b, sidx_b, rows_sb)

                @pl.when(jj < (NCHUNK - 1) // 2 - 1)
                def _():
                    sistart(j0 + 3, sidx_b, sem_ib)

            gw(NCHUNK - 1, rows_a, sem_a)
            siwait(sidx_a, sem_ia)
            scatter(NCHUNK - 1, rows_a, sidx_a, rows_sa)

            plsc.subcore_barrier()

            @pl.when(s < 10)
            def _(v=v, out_hbm=out_hbm):
                pltpu.sync_copy(
                    acc.at[pl.ds(s * RB, RB)],
                    out_hbm.at[c, pl.ds(s * RB, RB)],
                )

            if sidecar:
                @pl.when(s == 10)
                def _(v=v, outs_hbm=outs_hbm):
                    pltpu.sync_copy(accs, outs_hbm.at[c, 0])

    return functools.partial(
        pl.kernel, out_type=out_type, mesh=_mesh(), scratch_types=scratch,
        compiler_params=pltpu.CompilerParams(needs_layout_passes=not sidecar),
    )(body)


def _full(shape):
    return pl.BlockSpec(shape, lambda i: tuple(0 for _ in shape))


@functools.lru_cache(maxsize=None)
def _tc_prep(kind):
    def body(cv_ref, ce_ref, x_ref, xm_ref, sc_ref):
        if kind == "g":
            deg = ce_ref[0] + ce_ref[1] + 1.0
            s0 = lax.rsqrt(deg)
            s1 = s0 * s0
        else:
            dv = jnp.maximum(cv_ref[0] + cv_ref[1], 1.0)
            de = jnp.maximum(ce_ref[0] + ce_ref[1], 1.0)
            s0 = lax.rsqrt(dv)
            s1 = 1.0 / de
        xm_ref[:] = x_ref[:] * s0
        sc_ref[:] = jnp.concatenate(
            [s0, s1, jnp.zeros((RB, 14), _f32)], axis=1
        )

    return pl.pallas_call(
        body,
        grid=(NB,),
        in_specs=[
            pl.BlockSpec((2, RB, 1), lambda i: (0, i, 0)),
            pl.BlockSpec((2, RB, 1), lambda i: (0, i, 0)),
            pl.BlockSpec((RB, F), lambda i: (i, 0)),
        ],
        out_specs=[
            pl.BlockSpec((RB, F), lambda i: (i, 0)),
            pl.BlockSpec((RB, 16), lambda i: (i, 0)),
        ],
        out_shape=[
            jax.ShapeDtypeStruct((N, F), _f32),
            jax.ShapeDtypeStruct((N, 16), _f32),
        ],
    )


@functools.lru_cache(maxsize=None)
def _tc_row(kind):
    def body(*refs):
        if kind in ("prep_g", "prep_h"):
            c_ref, out_ref = refs
            acc = None
        else:
            a_ref, c_ref, out_ref = refs
            acc = a_ref[0] + a_ref[1]
        if kind == "prep_g":
            out_ref[...] = lax.rsqrt(c_ref[0] + c_ref[1] + 1.0)
        elif kind == "prep_h":
            out_ref[...] = lax.rsqrt(jnp.maximum(c_ref[0] + c_ref[1], 1.0))
        elif kind == "mid_h":
            out_ref[...] = acc / jnp.maximum(c_ref[0] + c_ref[1], 1.0)
        elif kind == "fin_h":
            out_ref[...] = acc * lax.rsqrt(jnp.maximum(c_ref[0] + c_ref[1], 1.0))
        elif kind == "fin_g":
            nrm = lax.rsqrt(c_ref[0] + c_ref[1] + 1.0)
            out_ref[...] = (acc + nrm) * nrm

    n_in = 1 if kind in ("prep_g", "prep_h") else 2
    return pl.pallas_call(
        body,
        in_specs=[pl.BlockSpec((2, 1, N), lambda: (0, 0, 0))] * n_in,
        out_specs=pl.BlockSpec((1, N), lambda: (0, 0)),
        out_shape=jax.ShapeDtypeStruct((1, N), _f32),
    )


@functools.lru_cache(maxsize=None)
def _tc_accscale(fw, col):
    def body(acc_ref, sc_ref, out_ref):
        out_ref[:] = (acc_ref[0] + acc_ref[1]) * sc_ref[:, col:col + 1]

    return pl.pallas_call(
        body,
        grid=(NB,),
        in_specs=[
            pl.BlockSpec((2, RB, fw), lambda i: (0, i, 0)),
            pl.BlockSpec((RB, 16), lambda i: (i, 0)),
        ],
        out_specs=pl.BlockSpec((RB, fw), lambda i: (i, 0)),
        out_shape=jax.ShapeDtypeStruct((N, fw), _f32),
    )


@functools.lru_cache(maxsize=None)
def _tc_block2(with_self):
    def body(*refs):
        if with_self:
            (acc_ref, xm_ref, sc_ref, s1_ref, w1_ref, b1_ref, w2_ref,
             b2_ref, yn_ref) = refs
            base = acc_ref[0] + acc_ref[1] + xm_ref[:]
        else:
            (acc_ref, sc_ref, s1_ref, w1_ref, b1_ref, w2_ref,
             b2_ref, yn_ref) = refs
            base = acc_ref[0] + acc_ref[1]
        s0 = sc_ref[:, 0:1]
        sm = base * s0
        h = jnp.dot(sm, w1_ref[:], preferred_element_type=_f32)
        h = jnp.maximum(h + s1_ref[:] * b1_ref[:], 0.0)
        y = jnp.dot(h, w2_ref[:], preferred_element_type=_f32) + b2_ref[:]
        yn_ref[:] = jnp.concatenate(
            [y * s0, jnp.zeros((RB, F - DO), _f32)], axis=1
        )

    in_specs = [pl.BlockSpec((2, RB, F), lambda i: (0, i, 0))]
    if with_self:
        in_specs.append(pl.BlockSpec((RB, F), lambda i: (i, 0)))
    in_specs += [
        pl.BlockSpec((RB, 16), lambda i: (i, 0)),
        pl.BlockSpec((RB, 1), lambda i: (i, 0)),
        _full((F, HID)),
        _full((1, HID)),
        _full((HID, DO)),
        _full((1, DO)),
    ]
    return pl.pallas_call(
        body,
        grid=(NB,),
        in_specs=in_specs,
        out_specs=pl.BlockSpec((RB, F), lambda i: (i, 0)),
        out_shape=jax.ShapeDtypeStruct((N, F), _f32),
    )


@functools.lru_cache(maxsize=None)
def _tc_attn(with_self):
    def body(*refs):
        if with_self:
            (acc_ref, yn_ref, sc_ref, wa, ba, wb, bb, wc, bc, wo, bo, g, b,
             out_ref, m_ref, z_ref, a64_ref) = refs
            base = acc_ref[0] + acc_ref[1] + yn_ref[:]
        else:
            (acc_ref, sc_ref, wa, ba, wb, bb, wc, bc, wo, bo, g, b,
             out_ref, m_ref, z_ref, a64_ref) = refs
            base = acc_ref[0] + acc_ref[1]
        i = pl.program_id(0)

        @pl.when(i == 0)
        def _():
            m_ref[...] = jnp.full((1, 1), -1e30, _f32)
            z_ref[...] = jnp.zeros((1, 1), _f32)
            a64_ref[...] = jnp.zeros((1, DO), _f32)

        xb = base[:, :DO] * sc_ref[:, 0:1]
        a = jnp.tanh(jnp.dot(xb, wa[:], preferred_element_type=_f32) + ba[:])
        gt = jax.nn.sigmoid(
            jnp.dot(xb, wb[:], preferred_element_type=_f32) + bb[:]
        )
        sco = jnp.dot(a * gt, wc[:], preferred_element_type=_f32) + bc[:]

        m_old = m_ref[...]
        m_new = jnp.maximum(m_old, jnp.max(sco))
        corr = jnp.exp(m_old - m_new)
        p = jnp.exp(sco - m_new[0, 0])
        z_ref[...] = z_ref[...] * corr + jnp.sum(p)
        a64_ref[...] = a64_ref[...] * corr + jnp.sum(
            p * xb, axis=0, keepdims=True
        )
        m_ref[...] = m_new

        @pl.when(i == NB - 1)
        def _():
            gf = a64_ref[...] / z_ref[0, 0]
            g2 = jnp.dot(gf, wo[:], preferred_element_type=_f32) + bo[:]
            mu = jnp.mean(g2)
            var = jnp.mean((g2 - mu) ** 2)
            out_ref[...] = (g2 - mu) * lax.rsqrt(var + 1e-5) * g[:] + b[:]

    in_specs = [pl.BlockSpec((2, RB, F), lambda i: (0, i, 0))]
    if with_self:
        in_specs.append(pl.BlockSpec((RB, F), lambda i: (i, 0)))
    in_specs += [
        pl.BlockSpec((RB, 16), lambda i: (i, 0)),
        _full((DO, HID)),
        _full((1, HID)),
        _full((DO, HID)),
        _full((1, HID)),
        _full((HID, 1)),
        _full((1, 1)),
        _full((DO, DO)),
        _full((1, DO)),
        _full((1, DO)),
        _full((1, DO)),
    ]
    return pl.pallas_call(
        body,
        grid=(NB,),
        in_specs=in_specs,
        out_specs=pl.BlockSpec((1, DO), lambda i: (0, 0)),
        out_shape=jax.ShapeDtypeStruct((1, DO), _f32),
        scratch_shapes=[
            pltpu.VMEM((1, 1), _f32),
            pltpu.VMEM((1, 1), _f32),
            pltpu.VMEM((1, DO), _f32),
        ],
    )


def _final_body(x_ref, g_ref, b_ref, wf_ref, bf_ref, out_ref):
    x = x_ref[:]
    mu = jnp.mean(x)
    var = jnp.mean((x - mu) ** 2)
    xn = (x - mu) * lax.rsqrt(var + 1e-5) * g_ref[:] + b_ref[:]
    out_ref[:] = jnp.dot(xn, wf_ref[:], preferred_element_type=_f32) + bf_ref[:]


@functools.lru_cache(maxsize=None)
def _tc_final():
    return pl.pallas_call(
        _final_body,
        out_shape=jax.ShapeDtypeStruct((1, 10), _f32),
    )


def kernel(X_H0, X_H1, X_H2, X_G0, X_G1, X_G2, hg0_idx, hg1_idx, hg2_idx,
           g0_idx, g1_idx, g2_idx, W_h1, b_h1, W_h2, b_h2, W_g1, b_g1,
           W_g2, b_g2, Wa, ba, Wb, bb, Wc, bc, Wo, bo, ln1_g, ln1_b,
           ln2_g, ln2_b, Wf, bf):
    zF = jnp.zeros((N, F), _f32)
    z1 = jnp.zeros((N,), _f32)
    z2N = jnp.zeros((2 * N,), _f32)
    onesCH = jnp.ones((CH,), _f32)

    b1h = b_h1[None, :]
    b1g = b_g1[None, :]
    b2h = b_h2[None, :]
    b2g = b_g2[None, :]
    attn_w = (Wa, ba[None, :], Wb, bb[None, :], Wc, bc[None, :],
              Wo, bo[None, :], ln1_g[None, :], ln1_b[None, :])

    deg_k = _sc_degree_kernel()

    views = (hg0_idx, hg1_idx, hg2_idx, g0_idx, g1_idx, g2_idx)
    Xs = (X_H0, X_H1, X_H2, X_G0, X_G1, X_G2)
    deg_in = []
    idxf = []
    for idx in views:
        i0, i1 = idx[0], idx[1]
        deg_in += [
            i0.reshape(NW, NCHUNK, CH), (i1 + N).reshape(NW, NCHUNK, CH)
        ]
        idxf.append((
            i0.reshape(NW, EPW), i1.reshape(NW, EPW),
            i0.reshape(NW * NCHUNK, CH), i1.reshape(NW * NCHUNK, CH),
        ))
    cnts = deg_k(*deg_in, onesCH, z2N)

    cv_r = [cnts[v][:, :, :N] for v in range(6)]
    ce_r = [cnts[v][:, :, N:] for v in range(6)]
    xm, sc, xs = [], [], []
    for v in range(6):
        cv_c = cv_r[v].reshape(NC, N, 1)
        ce_c = ce_r[v].reshape(NC, N, 1)
        if v < 3:
            xs.append(_tc_row("prep_h")(cv_r[v]).reshape(N))
            xm_v, sc_v = _tc_prep("h")(cv_c, ce_c, Xs[v])
        else:
            xs.append(_tc_row("prep_g")(ce_r[v]).reshape(N))
            xm_v, sc_v = _tc_prep("g")(cv_c, ce_c, Xs[v])
        xm.append(xm_v)
        sc.append(sc_v)

    w1_in = []
    for v in range(6):
        w1_in += [xm[v], xs[v], idxf[v][0], idxf[v][3]]
    w1 = _sc_segsum_kernel(6, True)(*w1_in, zF, z1)
    accm1, accs1 = w1[:6], w1[6:]

    hem = [_tc_accscale(F, 1)(accm1[v], sc[v]) for v in range(3)]
    hes = [_tc_row("mid_h")(accs1[v], ce_r[v]).reshape(N) for v in range(3)]
    w2_in = []
    for v in range(3):
        w2_in += [hem[v], hes[v], idxf[v][1], idxf[v][2]]
    w2 = _sc_segsum_kernel(3, True)(*w2_in, zF, z1)
    accm2, accs2 = w2[:3], w2[3:]

    yn = []
    for v in range(3):
        s1 = _tc_row("fin_h")(accs2[v], cv_r[v]).reshape(N, 1)
        yn.append(_tc_block2(False)(accm2[v], sc[v], s1, W_h1, b1h,
                                    W_h2, b2h))
    for v in range(3, 6):
        s1 = _tc_row("fin_g")(accs1[v], ce_r[v]).reshape(N, 1)
        yn.append(_tc_block2(True)(accm1[v], xm[v], sc[v], s1, W_g1, b1g,
                                   W_g2, b2g))

    w3_in = []
    for v in range(6):
        w3_in += [yn[v], idxf[v][0], idxf[v][3]]
    acc3 = _sc_segsum_kernel(6, False)(*w3_in, zF)

    he2 = [_tc_accscale(F, 1)(acc3[v], sc[v]) for v in range(3)]
    w4_in = []
    for v in range(3):
        w4_in += [he2[v], idxf[v][1], idxf[v][2]]
    acc4 = _sc_segsum_kernel(3, False)(*w4_in, zF)

    gvecs = []
    for v in range(3):
        gvecs.append(_tc_attn(False)(acc4[v], sc[v], *attn_w))
    for v in range(3, 6):
        gvecs.append(_tc_attn(True)(acc3[v], yn[v], sc[v], *attn_w))

    gcat = jnp.concatenate(gvecs, axis=1)
    return _tc_final()(
        gcat, ln2_g[None, :], ln2_b[None, :], Wf, bf[None, :]
    )


# speedup vs baseline: 1.0604x; 1.0604x over previous
import functools

import jax
import jax.numpy as jnp
from jax import lax
from jax.experimental import pallas as pl
from jax.experimental.pallas import tpu as pltpu
from jax.experimental.pallas import tpu_sc as plsc

N = 10000
E = 320000
F = 128
HID = 256
DO = 64
NC, NS = 2, 16
NW = NC * NS
EPW = E // NW
CH = 80
NCHUNK = EPW // CH
RB = 1000
NB = N // RB

_f32 = jnp.float32


def _mesh():
    return plsc.VectorSubcoreMesh(
        core_axis_name="c", subcore_axis_name="s", num_cores=NC, num_subcores=NS
    )


@functools.lru_cache(maxsize=None)
def _sc_degree_kernel():
    @functools.partial(
        pl.kernel,
        out_type=[jax.ShapeDtypeStruct((NC, 1, 2 * N), _f32)] * 6,
        mesh=_mesh(),
        scratch_types=(
            [pltpu.VMEM_SHARED((2 * N,), _f32)] * 6
            + [
                pltpu.VMEM((CH,), _f32),
                pltpu.VMEM((NCHUNK, CH), jnp.int32),
                pltpu.VMEM((NCHUNK, CH), jnp.int32),
                pltpu.SemaphoreType.DMA,
            ]
        ),
    )
    def k(*refs):
        idx_hbms = refs[0:12]
        ones_hbm, z_hbm = refs[12], refs[13]
        outs = refs[14:20]
        accs = refs[20:26]
        ones, ia_b, ib_b, sem = refs[26:30]
        c = lax.axis_index("c")
        s = lax.axis_index("s")
        wid = s * NC + c

        for v in range(6):
            @pl.when(s == v)
            def _(v=v):
                pltpu.sync_copy(z_hbm, accs[v])

        pltpu.sync_copy(ones_hbm, ones)
        plsc.subcore_barrier()

        for v in range(6):
            acc = accs[v]
            pltpu.sync_copy(idx_hbms[2 * v].at[wid], ia_b)
            pltpu.sync_copy(idx_hbms[2 * v + 1].at[wid], ib_b)

            def _wait_pair():
                pltpu.make_async_copy(ones, acc.at[ia_b.at[0]], sem).wait()
                pltpu.make_async_copy(ones, acc.at[ib_b.at[0]], sem).wait()

            @pl.loop(0, NCHUNK)
            def _(j, acc=acc, _wait_pair=_wait_pair):
                pltpu.async_copy(ones, acc.at[ia_b.at[j]], sem, add=True)
                pltpu.async_copy(ones, acc.at[ib_b.at[j]], sem, add=True)

                @pl.when(j > 0)
                def _():
                    _wait_pair()

            _wait_pair()

        plsc.subcore_barrier()

        for v in range(6):
            @pl.when(s == v)
            def _(v=v):
                pltpu.sync_copy(accs[v], outs[v].at[c, 0])

    return k


@functools.lru_cache(maxsize=None)
def _sc_segsum_kernel(nv, sidecar, fw_out=F):
    out_type = [jax.ShapeDtypeStruct((NC, N, fw_out), _f32)] * nv
    scratch = [
        pltpu.VMEM_SHARED((N, fw_out), _f32),
        pltpu.VMEM((CH, F), _f32),
        pltpu.VMEM((CH, F), _f32),
        pltpu.VMEM((EPW,), jnp.int32),
        pltpu.VMEM((1, CH), jnp.int32),
        pltpu.VMEM((1, CH), jnp.int32),
        pltpu.SemaphoreType.DMA,
        pltpu.SemaphoreType.DMA,
        pltpu.SemaphoreType.DMA,
        pltpu.SemaphoreType.DMA,
    ]
    if sidecar:
        out_type += [jax.ShapeDtypeStruct((NC, 1, N), _f32)] * nv
    if len(out_type) == 1:
        out_type = out_type[0]
    if sidecar:
        scratch += [
            pltpu.VMEM_SHARED((N,), _f32),
            pltpu.VMEM((CH,), _f32),
            pltpu.VMEM((CH,), _f32),
            pltpu.VMEM((N,), _f32),
        ]
    npv = 4 if sidecar else 3

    def body(*refs):
        n_in = npv * nv + 1 + (1 if sidecar else 0)
        ins = refs[0:n_in]
        outs = refs[n_in:n_in + (2 if sidecar else 1) * nv]
        scr = refs[n_in + len(outs):]
        if sidecar:
            (acc, rows_a, rows_b, gidx, sidx_a, sidx_b,
             sem_a, sem_b, sem_ia, sem_ib,
             accs, rows_sa, rows_sb, xs_v) = scr
        else:
            (acc, rows_a, rows_b, gidx, sidx_a, sidx_b,
             sem_a, sem_b, sem_ia, sem_ib) = scr
            rows_sa = rows_sb = xs_v = accs = None
        c = lax.axis_index("c")
        s = lax.axis_index("s")
        wid = s * NC + c
        srow0 = wid * NCHUNK

        z_hbm = ins[npv * nv]
        z1_hbm = ins[npv * nv + 1] if sidecar else None

        def gstart(xm_hbm, j, rows, sem):
            pltpu.async_copy(
                xm_hbm.at[gidx.at[pl.ds(j * CH, CH)]], rows, sem
            )

        def gwait(xm_hbm, j, rows, sem):
            pltpu.make_async_copy(
                xm_hbm.at[gidx.at[pl.ds(j * CH, CH)]], rows, sem
            ).wait()

        for v in range(nv):
            if sidecar:
                xm_hbm = ins[4 * v + 0]
                xs_hbm = ins[4 * v + 1]
                gids_hbm = ins[4 * v + 2]
                sids_hbm = ins[4 * v + 3]
                out_hbm = outs[v]
                outs_hbm = outs[nv + v]
            else:
                xm_hbm = ins[3 * v + 0]
                gids_hbm = ins[3 * v + 1]
                sids_hbm = ins[3 * v + 2]
                out_hbm = outs[v]
                outs_hbm = None

            @pl.when(s < 10)
            def _(v=v):
                pltpu.sync_copy(
                    z_hbm.at[pl.ds(s * RB, RB)], acc.at[pl.ds(s * RB, RB)]
                )

            if sidecar:
                @pl.when(s == 10)
                def _(v=v):
                    pltpu.sync_copy(z1_hbm, accs)

                pltpu.sync_copy(xs_hbm, xs_v)

            pltpu.sync_copy(gids_hbm.at[wid], gidx)
            plsc.subcore_barrier()

            def sistart(j, sbuf, sem, sids_hbm=sids_hbm):
                pltpu.async_copy(
                    sids_hbm.at[pl.ds(srow0 + j, 1)], sbuf, sem
                )

            def siwait(sbuf, sem, sids_hbm=sids_hbm):
                pltpu.make_async_copy(
                    sids_hbm.at[pl.ds(srow0, 1)], sbuf, sem
                ).wait()

            def scatter(j, rows, sbuf, rows_s):
                if sidecar:
                    for r in range(CH // 16):
                        gi = gidx[pl.ds(j * CH + r * 16, 16)]
                        rows_s[pl.ds(r * 16, 16)] = plsc.load_gather(
                            xs_v, [gi]
                        )
                src = rows if fw_out == F else rows.at[:, pl.ds(0, fw_out)]
                pltpu.sync_copy(src, acc.at[sbuf.at[0]], add=True)
                if sidecar:
                    pltpu.sync_copy(rows_s, accs.at[sbuf.at[0]], add=True)

            gs = functools.partial(gstart, xm_hbm)
            gw = functools.partial(gwait, xm_hbm)

            sistart(0, sidx_a, sem_ia)
            sistart(1, sidx_b, sem_ib)
            gs(0, rows_a, sem_a)

            @pl.loop(0, (NCHUNK - 1) // 2)
            def _(jj, gs=gs, gw=gw, sistart=sistart, siwait=siwait,
                  scatter=scatter):
                j0 = 2 * jj
                gs(j0 + 1, rows_b, sem_b)
                gw(j0, rows_a, sem_a)
                siwait(sidx_a, sem_ia)
                scatter(j0, rows_a, sidx_a, rows_sa)
                sistart(j0 + 2, sidx_a, sem_ia)
                gs(j0 + 2, rows_a, sem_a)
                gw(j0 + 1, rows_b, sem_b)
                siwait(sidx_b, sem_ib)
                scatter(j0 + 1, rows_b, sidx_b, rows_sb)

                @pl.when(jj < (NCHUNK - 1) // 2 - 1)
                def _():
                    sistart(j0 + 3, sidx_b, sem_ib)

            gw(NCHUNK - 1, rows_a, sem_a)
            siwait(sidx_a, sem_ia)
            scatter(NCHUNK - 1, rows_a, sidx_a, rows_sa)

            plsc.subcore_barrier()

            @pl.when(s < 10)
            def _(v=v, out_hbm=out_hbm):
                pltpu.sync_copy(
                    acc.at[pl.ds(s * RB, RB)],
                    out_hbm.at[c, pl.ds(s * RB, RB)],
                )

            if sidecar:
                @pl.when(s == 10)
                def _(v=v, outs_hbm=outs_hbm):
                    pltpu.sync_copy(accs, outs_hbm.at[c, 0])

    return functools.partial(
        pl.kernel, out_type=out_type, mesh=_mesh(), scratch_types=scratch,
        compiler_params=pltpu.CompilerParams(needs_layout_passes=not sidecar),
    )(body)


def _full(shape):
    return pl.BlockSpec(shape, lambda i: tuple(0 for _ in shape))


@functools.lru_cache(maxsize=None)
def _tc_prep(kind):
    def body(cv_ref, ce_ref, x_ref, xm_ref, sc_ref):
        if kind == "g":
            deg = ce_ref[0] + ce_ref[1] + 1.0
            s0 = lax.rsqrt(deg)
            s1 = s0 * s0
        else:
            dv = jnp.maximum(cv_ref[0] + cv_ref[1], 1.0)
            de = jnp.maximum(ce_ref[0] + ce_ref[1], 1.0)
            s0 = lax.rsqrt(dv)
            s1 = 1.0 / de
        xm_ref[:] = x_ref[:] * s0
        sc_ref[:] = jnp.concatenate(
            [s0, s1, jnp.zeros((RB, 14), _f32)], axis=1
        )

    return pl.pallas_call(
        body,
        grid=(NB,),
        in_specs=[
            pl.BlockSpec((2, RB, 1), lambda i: (0, i, 0)),
            pl.BlockSpec((2, RB, 1), lambda i: (0, i, 0)),
            pl.BlockSpec((RB, F), lambda i: (i, 0)),
        ],
        out_specs=[
            pl.BlockSpec((RB, F), lambda i: (i, 0)),
            pl.BlockSpec((RB, 16), lambda i: (i, 0)),
        ],
        out_shape=[
            jax.ShapeDtypeStruct((N, F), _f32),
            jax.ShapeDtypeStruct((N, 16), _f32),
        ],
    )


@functools.lru_cache(maxsize=None)
def _tc_row(kind):
    def body(*refs):
        if kind in ("prep_g", "prep_h"):
            c_ref, out_ref = refs
            acc = None
        else:
            a_ref, c_ref, out_ref = refs
            acc = a_ref[0] + a_ref[1]
        if kind == "prep_g":
            out_ref[...] = lax.rsqrt(c_ref[0] + c_ref[1] + 1.0)
        elif kind == "prep_h":
            out_ref[...] = lax.rsqrt(jnp.maximum(c_ref[0] + c_ref[1], 1.0))
        elif kind == "mid_h":
            out_ref[...] = acc / jnp.maximum(c_ref[0] + c_ref[1], 1.0)
        elif kind == "fin_h":
            out_ref[...] = acc * lax.rsqrt(jnp.maximum(c_ref[0] + c_ref[1], 1.0))
        elif kind == "fin_g":
            nrm = lax.rsqrt(c_ref[0] + c_ref[1] + 1.0)
            out_ref[...] = (acc + nrm) * nrm

    n_in = 1 if kind in ("prep_g", "prep_h") else 2
    return pl.pallas_call(
        body,
        in_specs=[pl.BlockSpec((2, 1, N), lambda: (0, 0, 0))] * n_in,
        out_specs=pl.BlockSpec((1, N), lambda: (0, 0)),
        out_shape=jax.ShapeDtypeStruct((1, N), _f32),
    )


@functools.lru_cache(maxsize=None)
def _tc_accscale(fw, col, pad):
    def body(acc_ref, sc_ref, out_ref):
        r = (acc_ref[0] + acc_ref[1]) * sc_ref[:, col:col + 1]
        if pad:
            r = jnp.concatenate([r, jnp.zeros((RB, F - fw), _f32)], axis=1)
        out_ref[:] = r

    return pl.pallas_call(
        body,
        grid=(NB,),
        in_specs=[
            pl.BlockSpec((2, RB, fw), lambda i: (0, i, 0)),
            pl.BlockSpec((RB, 16), lambda i: (i, 0)),
        ],
        out_specs=pl.BlockSpec((RB, F if pad else fw), lambda i: (i, 0)),
        out_shape=jax.ShapeDtypeStruct((N, F if pad else fw), _f32),
    )


@functools.lru_cache(maxsize=None)
def _tc_block2(with_self):
    def body(*refs):
        if with_self:
            (acc_ref, xm_ref, sc_ref, s1_ref, w1_ref, b1_ref, w2_ref,
             b2_ref, yn_ref) = refs
            base = acc_ref[0] + acc_ref[1] + xm_ref[:]
        else:
            (acc_ref, sc_ref, s1_ref, w1_ref, b1_ref, w2_ref,
             b2_ref, yn_ref) = refs
            base = acc_ref[0] + acc_ref[1]
        s0 = sc_ref[:, 0:1]
        sm = base * s0
        h = jnp.dot(sm, w1_ref[:], preferred_element_type=_f32)
        h = jnp.maximum(h + s1_ref[:] * b1_ref[:], 0.0)
        y = jnp.dot(h, w2_ref[:], preferred_element_type=_f32) + b2_ref[:]
        yn_ref[:] = jnp.concatenate(
            [y * s0, jnp.zeros((RB, F - DO), _f32)], axis=1
        )

    in_specs = [pl.BlockSpec((2, RB, F), lambda i: (0, i, 0))]
    if with_self:
        in_specs.append(pl.BlockSpec((RB, F), lambda i: (i, 0)))
    in_specs += [
        pl.BlockSpec((RB, 16), lambda i: (i, 0)),
        pl.BlockSpec((RB, 1), lambda i: (i, 0)),
        _full((F, HID)),
        _full((1, HID)),
        _full((HID, DO)),
        _full((1, DO)),
    ]
    return pl.pallas_call(
        body,
        grid=(NB,),
        in_specs=in_specs,
        out_specs=pl.BlockSpec((RB, F), lambda i: (i, 0)),
        out_shape=jax.ShapeDtypeStruct((N, F), _f32),
    )


@functools.lru_cache(maxsize=None)
def _tc_attn(with_self):
    def body(*refs):
        if with_self:
            (acc_ref, yn_ref, sc_ref, wa, ba, wb, bb, wc, bc, wo, bo, g, b,
             out_ref, m_ref, z_ref, a64_ref) = refs
            base = acc_ref[0] + acc_ref[1] + yn_ref[:]
        else:
            (acc_ref, sc_ref, wa, ba, wb, bb, wc, bc, wo, bo, g, b,
             out_ref, m_ref, z_ref, a64_ref) = refs
            base = acc_ref[0] + acc_ref[1]
        i = pl.program_id(0)

        @pl.when(i == 0)
        def _():
            m_ref[...] = jnp.full((1, 1), -1e30, _f32)
            z_ref[...] = jnp.zeros((1, 1), _f32)
            a64_ref[...] = jnp.zeros((1, DO), _f32)

        xb = base[:, :DO] * sc_ref[:, 0:1]
        a = jnp.tanh(jnp.dot(xb, wa[:], preferred_element_type=_f32) + ba[:])
        gt = jax.nn.sigmoid(
            jnp.dot(xb, wb[:], preferred_element_type=_f32) + bb[:]
        )
        sco = jnp.dot(a * gt, wc[:], preferred_element_type=_f32) + bc[:]

        m_old = m_ref[...]
        m_new = jnp.maximum(m_old, jnp.max(sco))
        corr = jnp.exp(m_old - m_new)
        p = jnp.exp(sco - m_new[0, 0])
        z_ref[...] = z_ref[...] * corr + jnp.sum(p)
        a64_ref[...] = a64_ref[...] * corr + jnp.sum(
            p * xb, axis=0, keepdims=True
        )
        m_ref[...] = m_new

        @pl.when(i == NB - 1)
        def _():
            gf = a64_ref[...] / z_ref[0, 0]
            g2 = jnp.dot(gf, wo[:], preferred_element_type=_f32) + bo[:]
            mu = jnp.mean(g2)
            var = jnp.mean((g2 - mu) ** 2)
            out_ref[...] = (g2 - mu) * lax.rsqrt(var + 1e-5) * g[:] + b[:]

    in_specs = [pl.BlockSpec((2, RB, F), lambda i: (0, i, 0))]
    if with_self:
        in_specs.append(pl.BlockSpec((RB, F), lambda i: (i, 0)))
    in_specs += [
        pl.BlockSpec((RB, 16), lambda i: (i, 0)),
        _full((DO, HID)),
        _full((1, HID)),
        _full((DO, HID)),
        _full((1, HID)),
        _full((HID, 1)),
        _full((1, 1)),
        _full((DO, DO)),
        _full((1, DO)),
        _full((1, DO)),
        _full((1, DO)),
    ]
    return pl.pallas_call(
        body,
        grid=(NB,),
        in_specs=in_specs,
        out_specs=pl.BlockSpec((1, DO), lambda i: (0, 0)),
        out_shape=jax.ShapeDtypeStruct((1, DO), _f32),
        scratch_shapes=[
            pltpu.VMEM((1, 1), _f32),
            pltpu.VMEM((1, 1), _f32),
            pltpu.VMEM((1, DO), _f32),
        ],
    )


def _final_body(x_ref, g_ref, b_ref, wf_ref, bf_ref, out_ref):
    x = x_ref[:]
    mu = jnp.mean(x)
    var = jnp.mean((x - mu) ** 2)
    xn = (x - mu) * lax.rsqrt(var + 1e-5) * g_ref[:] + b_ref[:]
    out_ref[:] = jnp.dot(xn, wf_ref[:], preferred_element_type=_f32) + bf_ref[:]


@functools.lru_cache(maxsize=None)
def _tc_final():
    return pl.pallas_call(
        _final_body,
        out_shape=jax.ShapeDtypeStruct((1, 10), _f32),
    )


def kernel(X_H0, X_H1, X_H2, X_G0, X_G1, X_G2, hg0_idx, hg1_idx, hg2_idx,
           g0_idx, g1_idx, g2_idx, W_h1, b_h1, W_h2, b_h2, W_g1, b_g1,
           W_g2, b_g2, Wa, ba, Wb, bb, Wc, bc, Wo, bo, ln1_g, ln1_b,
           ln2_g, ln2_b, Wf, bf):
    zF = jnp.zeros((N, F), _f32)
    z1 = jnp.zeros((N,), _f32)
    z2N = jnp.zeros((2 * N,), _f32)
    onesCH = jnp.ones((CH,), _f32)

    b1h = b_h1[None, :]
    b1g = b_g1[None, :]
    b2h = b_h2[None, :]
    b2g = b_g2[None, :]
    attn_w = (Wa, ba[None, :], Wb, bb[None, :], Wc, bc[None, :],
              Wo, bo[None, :], ln1_g[None, :], ln1_b[None, :])

    deg_k = _sc_degree_kernel()

    views = (hg0_idx, hg1_idx, hg2_idx, g0_idx, g1_idx, g2_idx)
    Xs = (X_H0, X_H1, X_H2, X_G0, X_G1, X_G2)
    deg_in = []
    idxf = []
    for idx in views:
        i0, i1 = idx[0], idx[1]
        deg_in += [
            i0.reshape(NW, NCHUNK, CH), (i1 + N).reshape(NW, NCHUNK, CH)
        ]
        idxf.append((
            i0.reshape(NW, EPW), i1.reshape(NW, EPW),
            i0.reshape(NW * NCHUNK, CH), i1.reshape(NW * NCHUNK, CH),
        ))
    cnts = deg_k(*deg_in, onesCH, z2N)

    cv_r = [cnts[v][:, :, :N] for v in range(6)]
    ce_r = [cnts[v][:, :, N:] for v in range(6)]
    xm, sc, xs = [], [], []
    for v in range(6):
        cv_c = cv_r[v].reshape(NC, N, 1)
        ce_c = ce_r[v].reshape(NC, N, 1)
        if v < 3:
            xs.append(_tc_row("prep_h")(cv_r[v]).reshape(N))
            xm_v, sc_v = _tc_prep("h")(cv_c, ce_c, Xs[v])
        else:
            xs.append(_tc_row("prep_g")(ce_r[v]).reshape(N))
            xm_v, sc_v = _tc_prep("g")(cv_c, ce_c, Xs[v])
        xm.append(xm_v)
        sc.append(sc_v)

    seg1s = _sc_segsum_kernel(1, True)
    seg1p = _sc_segsum_kernel(1, False)

    gvecs = []
    for v in range(3):
        accm1, accs1 = seg1s(xm[v], xs[v], idxf[v][0], idxf[v][3], zF, z1)
        hem = _tc_accscale(F, 1, False)(accm1, sc[v])
        hes = _tc_row("mid_h")(accs1, ce_r[v]).reshape(N)
        accm2, accs2 = seg1s(hem, hes, idxf[v][1], idxf[v][2], zF, z1)
        s1 = _tc_row("fin_h")(accs2, cv_r[v]).reshape(N, 1)
        yn = _tc_block2(False)(accm2, sc[v], s1, W_h1, b1h, W_h2, b2h)
        acc3 = seg1p(yn, idxf[v][0], idxf[v][3], zF)
        he2 = _tc_accscale(F, 1, False)(acc3, sc[v])
        acc4 = seg1p(he2, idxf[v][1], idxf[v][2], zF)
        gvecs.append(_tc_attn(False)(acc4, sc[v], *attn_w))

    for v in range(3, 6):
        accm1, accs1 = seg1s(xm[v], xs[v], idxf[v][0], idxf[v][3], zF, z1)
        s1 = _tc_row("fin_g")(accs1, ce_r[v]).reshape(N, 1)
        yn = _tc_block2(True)(accm1, xm[v], sc[v], s1, W_g1, b1g,
                              W_g2, b2g)
        acc2 = seg1p(yn, idxf[v][0], idxf[v][3], zF)
        gvecs.append(_tc_attn(True)(acc2, yn, sc[v], *attn_w))

    gcat = jnp.concatenate(gvecs, axis=1)
    return _tc_final()(
        gcat, ln2_g[None, :], ln2_b[None, :], Wf, bf[None, :]
    )
